# baseline (speedup 1.0000x reference)
import functools

import jax
import jax.numpy as jnp
from jax import lax
from jax.experimental import pallas as pl
from jax.experimental.pallas import tpu as pltpu
from jax.experimental.pallas import tpu_sc as plsc

_N = 100000
_G = 128
_ROWS = 4096

_NT = 32
_CHUNK = 3128
_LAST = _N - (_NT - 1) * _CHUNK


def _sc_partial_counts(batch):
    mesh = plsc.VectorSubcoreMesh(core_axis_name="c", subcore_axis_name="s")

    @functools.partial(
        pl.kernel,
        mesh=mesh,
        compiler_params=pltpu.CompilerParams(needs_layout_passes=False),
        out_type=jax.ShapeDtypeStruct((_NT, _G), jnp.int32),
        scratch_types=[
            pltpu.VMEM((_CHUNK,), jnp.int32),
            pltpu.VMEM((_G,), jnp.int32),
        ],
    )
    def k(batch_hbm, out_hbm, b_v, lub_v):
        c = lax.axis_index("c")
        s = lax.axis_index("s")
        wid = s * 2 + c
        lane = lax.broadcasted_iota(jnp.int32, (16,), 0)

        @pl.when(wid < _NT - 1)
        def _():
            pltpu.sync_copy(batch_hbm.at[pl.ds(wid * _CHUNK, _CHUNK)], b_v)

        @pl.when(wid == _NT - 1)
        def _():
            pltpu.sync_copy(batch_hbm.at[pl.ds((_NT - 1) * _CHUNK, _LAST)],
                            b_v.at[pl.ds(0, _LAST)])

        n = jnp.where(wid == _NT - 1, _LAST, _CHUNK)

        def per_vec(k8, _):
            g = lane + 16 * k8

            def step(carry):
                lo, hi = carry
                active = lo < hi
                mid = lax.shift_right_arithmetic(lo + hi, 1)
                v = plsc.load_gather(b_v, [jnp.minimum(mid, n - 1)])
                take = jnp.logical_and(active, v <= g)
                lo = jnp.where(take, mid + 1, lo)
                hi = jnp.where(jnp.logical_and(active, v > g), mid, hi)
                return lo, hi

            lo, _hi = lax.while_loop(
                lambda carry: jnp.any(carry[0] < carry[1]),
                step, (jnp.zeros((16,), jnp.int32), jnp.full((16,), 1, jnp.int32) * n))
            lub_v[pl.ds(k8 * 16, 16)] = lo
            return 0

        lax.fori_loop(0, _G // 16, per_vec, 0)
        pltpu.sync_copy(lub_v, out_hbm.at[wid])

    return k(batch)


def _scale_body(x_ref, b_ref, cnt_ref, o_ref):
    i = pl.program_id(0)
    ub = jnp.sum(cnt_ref[:, :].astype(jnp.float32), axis=0, keepdims=True)
    prev = pltpu.roll(ub, 1, axis=1)
    lane0 = lax.broadcasted_iota(jnp.int32, (1, _G), 1) == 0
    deg = ub - jnp.where(lane0, 0.0, prev)
    inv = lax.rsqrt(deg)[0, :]
    b = b_ref[pl.ds(i * _ROWS, _ROWS)]
    gid = lax.broadcasted_iota(jnp.int32, (_ROWS, _G), 1)
    eq = b[:, None] == gid
    scale = jnp.sum(jnp.where(eq, inv[None, :], 0.0), axis=1)
    o_ref[:, :] = x_ref[:, :] * scale[:, None]


def kernel(x, batch):
    b32 = batch.astype(jnp.int32)
    cnt = _sc_partial_counts(b32)
    grid_n = -(-_N // _ROWS)
    b_pad = jnp.pad(b32, (0, grid_n * _ROWS - _N))
    return pl.pallas_call(
        _scale_body,
        grid=(grid_n,),
        in_specs=[
            pl.BlockSpec((_ROWS, 512), lambda i: (i, 0)),
            pl.BlockSpec((grid_n * _ROWS,), lambda i: (0,)),
            pl.BlockSpec((_NT, _G), lambda i: (0, 0)),
        ],
        out_specs=pl.BlockSpec((_ROWS, 512), lambda i: (i, 0)),
        out_shape=jax.ShapeDtypeStruct((_N, 512), jnp.float32),
        compiler_params=pltpu.CompilerParams(
            dimension_semantics=("arbitrary",),
        ),
    )(x, b_pad, cnt)


# speedup vs baseline: 1.0067x; 1.0067x over previous
import functools

import jax
import jax.numpy as jnp
from jax import lax
from jax.experimental import pallas as pl
from jax.experimental.pallas import tpu as pltpu
from jax.experimental.pallas import tpu_sc as plsc

_N = 100000
_G = 128
_ROWS = 4352

_NT = 32
_CHUNK = 3128
_LAST = _N - (_NT - 1) * _CHUNK


def _sc_partial_counts(batch):
    mesh = plsc.VectorSubcoreMesh(core_axis_name="c", subcore_axis_name="s")

    @functools.partial(
        pl.kernel,
        mesh=mesh,
        compiler_params=pltpu.CompilerParams(needs_layout_passes=False),
        out_type=jax.ShapeDtypeStruct((_NT, _G), jnp.int32),
        scratch_types=[
            pltpu.VMEM((_CHUNK,), jnp.int32),
            pltpu.VMEM((_G,), jnp.int32),
        ],
    )
    def k(batch_hbm, out_hbm, b_v, lub_v):
        c = lax.axis_index("c")
        s = lax.axis_index("s")
        wid = s * 2 + c
        lane = lax.broadcasted_iota(jnp.int32, (16,), 0)

        @pl.when(wid < _NT - 1)
        def _():
            pltpu.sync_copy(batch_hbm.at[pl.ds(wid * _CHUNK, _CHUNK)], b_v)

        @pl.when(wid == _NT - 1)
        def _():
            pltpu.sync_copy(batch_hbm.at[pl.ds((_NT - 1) * _CHUNK, _LAST)],
                            b_v.at[pl.ds(0, _LAST)])

        n = jnp.where(wid == _NT - 1, _LAST, _CHUNK)

        def per_vec(k8, _):
            g = lane + 16 * k8

            def step(_, carry):
                lo, hi = carry
                active = lo < hi
                mid = lax.shift_right_arithmetic(lo + hi, 1)
                v = plsc.load_gather(b_v, [jnp.minimum(mid, n - 1)])
                take = jnp.logical_and(active, v <= g)
                lo = jnp.where(take, mid + 1, lo)
                hi = jnp.where(jnp.logical_and(active, v > g), mid, hi)
                return lo, hi

            lo, _hi = lax.fori_loop(
                0, 12,
                step, (jnp.zeros((16,), jnp.int32), jnp.full((16,), 1, jnp.int32) * n))
            lub_v[pl.ds(k8 * 16, 16)] = lo
            return 0

        lax.fori_loop(0, _G // 16, per_vec, 0)
        pltpu.sync_copy(lub_v, out_hbm.at[wid])

    return k(batch)


def _scale_body(x_ref, b_ref, cnt_ref, o_ref):
    i = pl.program_id(0)
    ub = jnp.sum(cnt_ref[:, :].astype(jnp.float32), axis=0, keepdims=True)
    prev = pltpu.roll(ub, 1, axis=1)
    lane0 = lax.broadcasted_iota(jnp.int32, (1, _G), 1) == 0
    deg = ub - jnp.where(lane0, 0.0, prev)
    inv = lax.rsqrt(deg)[0, :]
    b = b_ref[pl.ds(i * _ROWS, _ROWS)]
    gid = lax.broadcasted_iota(jnp.int32, (_ROWS, _G), 1)
    eq = b[:, None] == gid
    scale = jnp.sum(jnp.where(eq, inv[None, :], 0.0), axis=1)
    o_ref[:, :] = x_ref[:, :] * scale[:, None]


def kernel(x, batch):
    b32 = batch.astype(jnp.int32)
    cnt = _sc_partial_counts(b32)
    grid_n = -(-_N // _ROWS)
    b_pad = jnp.pad(b32, (0, grid_n * _ROWS - _N))
    return pl.pallas_call(
        _scale_body,
        grid=(grid_n,),
        in_specs=[
            pl.BlockSpec((_ROWS, 512), lambda i: (i, 0)),
            pl.BlockSpec((grid_n * _ROWS,), lambda i: (0,)),
            pl.BlockSpec((_NT, _G), lambda i: (0, 0)),
        ],
        out_specs=pl.BlockSpec((_ROWS, 512), lambda i: (i, 0)),
        out_shape=jax.ShapeDtypeStruct((_N, 512), jnp.float32),
        compiler_params=pltpu.CompilerParams(
            dimension_semantics=("arbitrary",),
        ),
    )(x, b_pad, cnt)


# speedup vs baseline: 1.0117x; 1.0050x over previous
import functools

import jax
import jax.numpy as jnp
from jax import lax
from jax.experimental import pallas as pl
from jax.experimental.pallas import tpu as pltpu
from jax.experimental.pallas import tpu_sc as plsc

_N = 100000
_G = 128
_ROWS = 4096

_NT = 32
_CHUNK = 3128
_LAST = _N - (_NT - 1) * _CHUNK


def _sc_partial_counts(batch):
    mesh = plsc.VectorSubcoreMesh(core_axis_name="c", subcore_axis_name="s")

    @functools.partial(
        pl.kernel,
        mesh=mesh,
        compiler_params=pltpu.CompilerParams(needs_layout_passes=False),
        out_type=jax.ShapeDtypeStruct((_NT, _G), jnp.int32),
        scratch_types=[
            pltpu.VMEM((_CHUNK,), jnp.int32),
            pltpu.VMEM((_G,), jnp.int32),
        ],
    )
    def k(batch_hbm, out_hbm, b_v, lub_v):
        c = lax.axis_index("c")
        s = lax.axis_index("s")
        wid = s * 2 + c
        lane = lax.broadcasted_iota(jnp.int32, (16,), 0)

        @pl.when(wid < _NT - 1)
        def _():
            pltpu.sync_copy(batch_hbm.at[pl.ds(wid * _CHUNK, _CHUNK)], b_v)

        @pl.when(wid == _NT - 1)
        def _():
            pltpu.sync_copy(batch_hbm.at[pl.ds((_NT - 1) * _CHUNK, _LAST)],
                            b_v.at[pl.ds(0, _LAST)])

        n = jnp.where(wid == _NT - 1, _LAST, _CHUNK)

        def per_vec(k8, _):
            g = lane + 16 * k8

            def step(_, carry):
                lo, hi = carry
                active = lo < hi
                mid = lax.shift_right_arithmetic(lo + hi, 1)
                v = plsc.load_gather(b_v, [jnp.minimum(mid, n - 1)])
                take = jnp.logical_and(active, v <= g)
                lo = jnp.where(take, mid + 1, lo)
                hi = jnp.where(jnp.logical_and(active, v > g), mid, hi)
                return lo, hi

            lo, _hi = lax.fori_loop(
                0, 12,
                step, (jnp.zeros((16,), jnp.int32), jnp.full((16,), 1, jnp.int32) * n))
            lub_v[pl.ds(k8 * 16, 16)] = lo
            return 0

        lax.fori_loop(0, _G // 16, per_vec, 0)
        pltpu.sync_copy(lub_v, out_hbm.at[wid])

    return k(batch)


def _scale_body(x_ref, b_ref, cnt_ref, o_ref):
    i = pl.program_id(0)
    ub = jnp.sum(cnt_ref[:, :].astype(jnp.float32), axis=0, keepdims=True)
    prev = pltpu.roll(ub, 1, axis=1)
    lane0 = lax.broadcasted_iota(jnp.int32, (1, _G), 1) == 0
    deg = ub - jnp.where(lane0, 0.0, prev)
    inv = lax.rsqrt(deg)[0, :]
    b = b_ref[pl.ds(i * _ROWS, _ROWS)]
    gid = lax.broadcasted_iota(jnp.int32, (_ROWS, _G), 1)
    eq = b[:, None] == gid
    scale = jnp.sum(jnp.where(eq, inv[None, :], 0.0), axis=1)
    o_ref[:, :] = x_ref[:, :] * scale[:, None]


def kernel(x, batch):
    b32 = batch.astype(jnp.int32)
    cnt = _sc_partial_counts(b32)
    grid_n = -(-_N // _ROWS)
    b_pad = jnp.pad(b32, (0, grid_n * _ROWS - _N))
    return pl.pallas_call(
        _scale_body,
        grid=(grid_n,),
        in_specs=[
            pl.BlockSpec((_ROWS, 512), lambda i: (i, 0)),
            pl.BlockSpec((grid_n * _ROWS,), lambda i: (0,)),
            pl.BlockSpec((_NT, _G), lambda i: (0, 0)),
        ],
        out_specs=pl.BlockSpec((_ROWS, 512), lambda i: (i, 0)),
        out_shape=jax.ShapeDtypeStruct((_N, 512), jnp.float32),
        compiler_params=pltpu.CompilerParams(
            dimension_semantics=("parallel",),
        ),
    )(x, b_pad, cnt)


# speedup vs baseline: 1.0145x; 1.0028x over previous
import functools

import jax
import jax.numpy as jnp
from jax import lax
from jax.experimental import pallas as pl
from jax.experimental.pallas import tpu as pltpu
from jax.experimental.pallas import tpu_sc as plsc

_N = 100000
_G = 128
_ROWS = 4096

_NT = 32
_CHUNK = 3128
_LAST = _N - (_NT - 1) * _CHUNK


def _sc_partial_counts(batch):
    mesh = plsc.VectorSubcoreMesh(core_axis_name="c", subcore_axis_name="s")

    @functools.partial(
        pl.kernel,
        mesh=mesh,
        compiler_params=pltpu.CompilerParams(needs_layout_passes=False),
        out_type=jax.ShapeDtypeStruct((_NT, _G), jnp.int32),
        scratch_types=[
            pltpu.VMEM((_CHUNK,), jnp.int32),
            pltpu.VMEM((_G,), jnp.int32),
        ],
    )
    def k(batch_hbm, out_hbm, b_v, lub_v):
        c = lax.axis_index("c")
        s = lax.axis_index("s")
        wid = s * 2 + c
        lane = lax.broadcasted_iota(jnp.int32, (16,), 0)

        @pl.when(wid < _NT - 1)
        def _():
            pltpu.sync_copy(batch_hbm.at[pl.ds(wid * _CHUNK, _CHUNK)], b_v)

        @pl.when(wid == _NT - 1)
        def _():
            pltpu.sync_copy(batch_hbm.at[pl.ds((_NT - 1) * _CHUNK, _LAST)],
                            b_v.at[pl.ds(0, _LAST)])

        n = jnp.where(wid == _NT - 1, _LAST, _CHUNK)

        def per_vec(k8, _):
            g0 = lane + 16 * k8
            g1 = g0 + (_G // 2)

            def one(g, carry):
                lo, hi = carry
                active = lo < hi
                mid = lax.shift_right_arithmetic(lo + hi, 1)
                v = plsc.load_gather(b_v, [jnp.minimum(mid, n - 1)])
                take = jnp.logical_and(active, v <= g)
                lo = jnp.where(take, mid + 1, lo)
                hi = jnp.where(jnp.logical_and(active, v > g), mid, hi)
                return lo, hi

            def step(_, carry):
                c0, c1 = carry
                return one(g0, c0), one(g1, c1)

            init = (jnp.zeros((16,), jnp.int32), jnp.full((16,), 1, jnp.int32) * n)
            (lo0, _h0), (lo1, _h1) = lax.fori_loop(
                0, 12,
                step, (init, init))
            lub_v[pl.ds(k8 * 16, 16)] = lo0
            lub_v[pl.ds(k8 * 16 + (_G // 2), 16)] = lo1
            return 0

        lax.fori_loop(0, _G // 32, per_vec, 0)
        pltpu.sync_copy(lub_v, out_hbm.at[wid])

    return k(batch)


def _scale_body(x_ref, b_ref, cnt_ref, o_ref):
    i = pl.program_id(0)
    ub = jnp.sum(cnt_ref[:, :].astype(jnp.float32), axis=0, keepdims=True)
    prev = pltpu.roll(ub, 1, axis=1)
    lane0 = lax.broadcasted_iota(jnp.int32, (1, _G), 1) == 0
    deg = ub - jnp.where(lane0, 0.0, prev)
    inv = lax.rsqrt(deg)[0, :]
    b = b_ref[pl.ds(i * _ROWS, _ROWS)]
    gid = lax.broadcasted_iota(jnp.int32, (_ROWS, _G), 1)
    eq = b[:, None] == gid
    scale = jnp.sum(jnp.where(eq, inv[None, :], 0.0), axis=1)
    o_ref[:, :] = x_ref[:, :] * scale[:, None]


def kernel(x, batch):
    b32 = batch.astype(jnp.int32)
    cnt = _sc_partial_counts(b32)
    grid_n = -(-_N // _ROWS)
    b_pad = jnp.pad(b32, (0, grid_n * _ROWS - _N))
    return pl.pallas_call(
        _scale_body,
        grid=(grid_n,),
        in_specs=[
            pl.BlockSpec((_ROWS, 512), lambda i: (i, 0)),
            pl.BlockSpec((grid_n * _ROWS,), lambda i: (0,)),
            pl.BlockSpec((_NT, _G), lambda i: (0, 0)),
        ],
        out_specs=pl.BlockSpec((_ROWS, 512), lambda i: (i, 0)),
        out_shape=jax.ShapeDtypeStruct((_N, 512), jnp.float32),
        compiler_params=pltpu.CompilerParams(
            dimension_semantics=("parallel",),
        ),
    )(x, b_pad, cnt)
